# baseline (speedup 1.0000x reference)
import functools
import jax
import jax.numpy as jnp
from jax import lax
from jax.experimental import pallas as pl
from jax.experimental.pallas import tpu as pltpu
from jax.experimental.pallas import tpu_sc as plsc

_HP = 16
_N = _HP * _HP
_NP = 49
_ROWS = _NP * _N
_K = 9
_D4 = 32
_NEG = -3.0e38

_NW = 32
_RPW = _ROWS // _NW
_NG = 25


def _sim_body(scal_ref, x_ref, fw_ref, fb_ref, comb_ref, xflat_ref, d_scr):
    p = pl.program_id(0)
    sigma = scal_ref[0, 0]
    alpha = scal_ref[0, 1]

    @pl.when(p == 0)
    def _():
        ni = lax.broadcasted_iota(jnp.int32, (_N, _N), 0)
        mi = lax.broadcasted_iota(jnp.int32, (_N, _N), 1)
        dr = (ni // _HP) - (mi // _HP)
        dc = (ni % _HP) - (mi % _HP)
        d2 = (dr * dr + dc * dc).astype(jnp.float32)
        d_scr[...] = jnp.exp(d2 * (-1.0 / (2.0 * sigma * sigma)))

    Dm = d_scr[...]
    xm = x_ref[0]
    ft = lax.dot_general(fw_ref[...], xm, (((1,), (0,)), ((), ())),
                         preferred_element_type=jnp.float32)
    ft = ft + fb_ref[...]
    nsq = jnp.sum(ft * ft, axis=0)[None, :]
    inv = 1.0 / jnp.maximum(jnp.sqrt(nsq), 1e-8)
    ftn = ft * inv
    S = lax.dot_general(ftn, ftn, (((0,), (0,)), ((), ())),
                        preferred_element_type=jnp.float32)
    comb_ref[...] = (alpha * S + (1.0 - alpha) * Dm)[None]
    xflat_ref[...] = ft.T[None]


def _proj_body(agg_ref, pw_ref, pb_ref, out_ref):
    y = lax.dot_general(pw_ref[...], agg_ref[0], (((1,), (1,)), ((), ())),
                        preferred_element_type=jnp.float32)
    out_ref[...] = (y + pb_ref[...])[None]


def _sc_mid_body(comb_hbm, xflat_hbm, params_hbm, out_hbm,
                 params_v, rowb, idxa, idxb, rowsg, outv, sem):
    wid = lax.axis_index("s") * 2 + lax.axis_index("c")
    base = wid * _RPW
    pltpu.sync_copy(params_hbm, params_v)

    def P(i):
        return params_v[i]

    iot = lax.iota(jnp.int32, 16)

    def grp_body(gi, carry):
        r0 = lax.min(gi * 16, _RPW - 16)
        g0 = base + r0
        pltpu.sync_copy(comb_hbm.at[pl.ds(g0, 16)], rowb)
        g_vec = iot + jnp.full((16,), g0, jnp.int32)
        n_vec = lax.rem(g_vec, _N)
        pbase_vec = g_vec - n_vec

        init = ([jnp.full((16,), _NEG, jnp.float32) for _ in range(_K)]
                + [jnp.zeros((16,), jnp.int32) for _ in range(_K)])

        def col_body(j, st):
            ts = list(st[:_K])
            tis = list(st[_K:])
            jv = jnp.full((16,), j, jnp.int32)
            v = plsc.load_gather(rowb, [iot, jv])
            vi = jv
            for s in range(_K):
                take = v > ts[s]
                nt = jnp.where(take, v, ts[s])
                nti = jnp.where(take, vi, tis[s])
                v = jnp.where(take, ts[s], v)
                vi = jnp.where(take, tis[s], vi)
                ts[s] = nt
                tis[s] = nti
            return tuple(ts) + tuple(tis)

        st = lax.fori_loop(0, _N, col_body, tuple(init))
        ts = st[:_K]
        tis = st[_K:]

        rn = n_vec >> 4
        cn = n_vec & 15
        wes = []
        for s in range(_K):
            ri = tis[s] >> 4
            ci = tis[s] & 15
            dr = rn - ri
            dc = cn - ci
            d2 = (dr * dr + dc * dc).astype(jnp.float32)
            sd = jnp.exp(d2 * P(0))
            sf = (ts[s] - sd * P(1)) * P(2)
            tot = P(19)
            for i in range(4):
                h = sf * P(3 + 2 * i) + sd * P(4 + 2 * i) + P(11 + i)
                h = h / (1.0 + jnp.exp(-h))
                tot = tot + h * P(15 + i)
            wes.append(1.0 / (1.0 + jnp.exp(-tot)))
        wsum = wes[0]
        for s in range(1, _K):
            wsum = wsum + wes[s]
        winv = 1.0 / (wsum + 1e-12)
        wns = [we * winv for we in wes]

        for s in range(5):
            idxa[pl.ds(s * 16, 16)] = pbase_vec + tis[s]
        for s in range(5, _K):
            idxb[pl.ds((s - 5) * 16, 16)] = pbase_vec + tis[s]
        cp_a = pltpu.async_copy(xflat_hbm.at[idxa], rowsg.at[pl.ds(0, 80)], sem)
        cp_b = pltpu.async_copy(xflat_hbm.at[idxb], rowsg.at[pl.ds(80, 64)], sem)
        cp_a.wait()
        cp_b.wait()

        rr = r0 + iot
        for d in range(_D4):
            dv = jnp.full((16,), d, jnp.int32)
            acc = wns[0] * plsc.load_gather(rowsg, [iot, dv])
            for s in range(1, _K):
                acc = acc + wns[s] * plsc.load_gather(rowsg, [iot + s * 16, dv])
            plsc.store_scatter(outv, [rr, dv], acc)
        return carry

    lax.fori_loop(0, _NG, grp_body, 0)
    pltpu.sync_copy(outv, out_hbm.at[pl.ds(base, _RPW)])


_sc_mid = functools.partial(
    pl.kernel,
    out_type=jax.ShapeDtypeStruct((_ROWS, _D4), jnp.float32),
    mesh=plsc.VectorSubcoreMesh(core_axis_name="c", subcore_axis_name="s"),
    compiler_params=pltpu.CompilerParams(use_tc_tiling_on_sc=False,
                                         needs_layout_passes=False),
    scratch_types=[
        pltpu.VMEM((24, 16), jnp.float32),
        pltpu.VMEM((16, _N), jnp.float32),
        pltpu.VMEM((80,), jnp.int32),
        pltpu.VMEM((64,), jnp.int32),
        pltpu.VMEM((144, _D4), jnp.float32),
        pltpu.VMEM((_RPW, _D4), jnp.float32),
        pltpu.SemaphoreType.DMA,
    ],
)(_sc_mid_body)


@jax.jit
def kernel(x_in, sigma, alpha, f_w, f_b, p_w, p_b, mlp_w1, mlp_b1, mlp_w2, mlp_b2):
    B, C, H, Wd = x_in.shape
    ws = 7
    scal = jnp.stack([sigma, alpha]).reshape(1, 2).astype(jnp.float32)
    pvec = jnp.concatenate([
        jnp.stack([
            -1.0 / (2.0 * sigma * sigma),
            1.0 - alpha,
            1.0 / alpha,
        ]),
        mlp_w1.reshape(-1), mlp_b1.reshape(-1),
        mlp_w2.reshape(-1), mlp_b2.reshape(-1),
        jnp.zeros((4,), jnp.float32),
    ]).astype(jnp.float32)
    params = jnp.tile(pvec.reshape(-1, 1), (1, 16))

    xp = x_in.reshape(C, ws, _HP, ws, _HP).transpose(1, 3, 0, 2, 4).reshape(_NP, C, _N)

    comb, xflat = pl.pallas_call(
        _sim_body,
        grid=(_NP,),
        in_specs=[
            pl.BlockSpec((1, 2), lambda p: (0, 0), memory_space=pltpu.SMEM),
            pl.BlockSpec((1, C, _N), lambda p: (p, 0, 0)),
            pl.BlockSpec((_D4, C), lambda p: (0, 0)),
            pl.BlockSpec((_D4, 1), lambda p: (0, 0)),
        ],
        out_specs=[
            pl.BlockSpec((1, _N, _N), lambda p: (p, 0, 0)),
            pl.BlockSpec((1, _N, _D4), lambda p: (p, 0, 0)),
        ],
        out_shape=[
            jax.ShapeDtypeStruct((_NP, _N, _N), jnp.float32),
            jax.ShapeDtypeStruct((_NP, _N, _D4), jnp.float32),
        ],
        scratch_shapes=[pltpu.VMEM((_N, _N), jnp.float32)],
    )(scal, xp, f_w, f_b.reshape(_D4, 1))

    out32 = _sc_mid(comb.reshape(_ROWS, _N), xflat.reshape(_ROWS, _D4), params)

    out = pl.pallas_call(
        _proj_body,
        grid=(_NP,),
        in_specs=[
            pl.BlockSpec((1, _N, _D4), lambda p: (p, 0, 0)),
            pl.BlockSpec((C, _D4), lambda p: (0, 0)),
            pl.BlockSpec((C, 1), lambda p: (0, 0)),
        ],
        out_specs=pl.BlockSpec((1, C, _N), lambda p: (p, 0, 0)),
        out_shape=jax.ShapeDtypeStruct((_NP, C, _N), jnp.float32),
    )(out32.reshape(_NP, _N, _D4), p_w, p_b.reshape(C, 1))

    out = out.reshape(ws, ws, C, _HP, _HP).transpose(2, 0, 3, 1, 4).reshape(B, C, H * Wd)
    return out
